# Initial kernel scaffold; baseline (speedup 1.0000x reference)
import jax
import jax.numpy as jnp
from jax.experimental import pallas as pl


def kernel(seq, table):
    raise NotImplementedError("write your pallas kernel here")


# speedup vs baseline: 1.6682x; 1.6682x over previous
import jax
import jax.numpy as jnp
from jax.experimental import pallas as pl

_ROWS = 1024


def _add_kernel(seq_ref, table_ref, out_ref):
    out_ref[...] = seq_ref[...] + table_ref[...]


def kernel(seq, table):
    b, c, d = seq.shape
    grid = (c // _ROWS, b)
    return pl.pallas_call(
        _add_kernel,
        grid=grid,
        in_specs=[
            pl.BlockSpec((1, _ROWS, d), lambda r, i: (i, r, 0)),
            pl.BlockSpec((_ROWS, d), lambda r, i: (r, 0)),
        ],
        out_specs=pl.BlockSpec((1, _ROWS, d), lambda r, i: (i, r, 0)),
        out_shape=jax.ShapeDtypeStruct(seq.shape, seq.dtype),
    )(seq, table)
